# Initial kernel scaffold; baseline (speedup 1.0000x reference)
import jax
import jax.numpy as jnp
from jax.experimental import pallas as pl


def kernel(hidden_states, gate_w, fc1_w, fc1_b, fc2_w, fc2_b, ln1_g, ln1_b, sp1_w, ln2_g, ln2_b, sp2_w):
    raise NotImplementedError("write your pallas kernel here")


# speedup vs baseline: 1.2879x; 1.2879x over previous
import functools

import jax
import jax.numpy as jnp
from jax.experimental import pallas as pl
from jax.experimental.pallas import tpu as pltpu

TOK = 4096
HID = 768
FFN = 3072
NE = 8
KSEL = 2
NG = 8
HGRP = NE // 2
BT = 256
NBLK = (TOK * KSEL) // BT + HGRP
NSLOT = NBLK * BT
KHALF = FFN // 2
GRID_LO, GRID_STEP = -1.2, 1.4 / (NG - 1)
INV_DEN = 0.5


def _route_meta(x, gate_w):
    logits = jax.lax.dot_general(x, gate_w, (((1,), (1,)), ((), ())))
    probs = jax.nn.softmax(logits.astype(jnp.float32), axis=1)
    topw, sel = jax.lax.top_k(probs, KSEL)
    topw = topw / jnp.sum(topw, axis=-1, keepdims=True)
    sel_f = sel.reshape(-1)
    w_f = topw.reshape(-1).astype(x.dtype)
    oneh = (sel_f[:, None] == jnp.arange(NE)[None, :]).astype(jnp.int32)
    rank = jnp.take_along_axis(jnp.cumsum(oneh, axis=0) - oneh,
                               sel_f[:, None], axis=1)[:, 0]
    counts = jnp.sum(oneh, axis=0)
    nblk_e = (counts + BT - 1) // BT

    def group_meta(goff):
        nb = nblk_e[goff:goff + HGRP]
        cum = jnp.cumsum(nb)
        bstart = jnp.concatenate([jnp.zeros((1,), cum.dtype), cum[:-1]])
        nact = cum[-1]
        bids = jnp.arange(NBLK)
        eid = jnp.minimum(jnp.searchsorted(cum, bids, side="right"),
                          HGRP - 1).astype(jnp.int32)
        return bstart.astype(jnp.int32), eid, nact.astype(jnp.int32)

    mlp_bstart, mlp_eid, mlp_nact = group_meta(0)
    kan_bstart, kan_eid, kan_nact = group_meta(HGRP)
    grp = sel_f // HGRP
    le = sel_f % HGRP
    bstart_pair = jnp.where(grp == 0, mlp_bstart[le], kan_bstart[le])
    gslot = grp * NSLOT + bstart_pair * BT + rank

    tok_of_pair = jnp.arange(TOK * KSEL, dtype=jnp.int32) // KSEL
    tok_map = jnp.zeros((2 * NSLOT,), jnp.int32).at[gslot].set(tok_of_pair)
    w_slot = jnp.zeros((2 * NSLOT,), x.dtype).at[gslot].set(w_f)
    return (tok_map, w_slot, gslot.reshape(TOK, KSEL),
            mlp_eid, mlp_nact, kan_eid, kan_nact)


def _layernorm(x, g, b):
    mu = jnp.mean(x, axis=-1, keepdims=True)
    var = jnp.mean((x - mu) ** 2, axis=-1, keepdims=True)
    return (x - mu) * jax.lax.rsqrt(var + 1e-5) * g[None, :] + b[None, :]


def _rswaf_gm(x):
    slabs = []
    for g in range(NG):
        t = jnp.tanh((x - (GRID_LO + g * GRID_STEP)) * INV_DEN)
        slabs.append(1.0 - t * t)
    return jnp.concatenate(slabs, axis=1)


def _mlp_body(eid_ref, nact_ref, xs_ref, w1_ref, b1_ref, w2_ref, b2_ref,
              ws_ref, out_ref):
    @pl.when(pl.program_id(0) < nact_ref[0])
    def _():
        x = xs_ref[...]
        h = jax.lax.dot_general(x, w1_ref[0],
                                (((1,), (1,)), ((), ()))) + b1_ref[0]
        h = 0.5 * h * (1.0 + jax.lax.erf(h * (2.0 ** -0.5)))
        y = jax.lax.dot_general(h, w2_ref[0],
                                (((1,), (1,)), ((), ()))) + b2_ref[0]
        out_ref[...] = y * ws_ref[...]


def _kan1_body(eid_ref, nact_ref, xs_ref, g_ref, b_ref, w_ref, out_ref):
    @pl.when(pl.program_id(0) < nact_ref[0])
    def _():
        xn = _layernorm(xs_ref[...], g_ref[0, 0], b_ref[0, 0])
        basis = _rswaf_gm(xn)
        out_ref[...] = jax.lax.dot_general(basis, w_ref[0],
                                           (((1,), (1,)), ((), ())))


def _kan2_body(chalf, eid_ref, nact_ref, ha_ref, hb_ref, g_ref, b_ref,
               w_ref, *rest):
    if chalf == 0:
        (out_ref,) = rest
    else:
        yprev_ref, ws_ref, out_ref = rest

    @pl.when(pl.program_id(0) < nact_ref[0])
    def _():
        h = jnp.concatenate([ha_ref[...], hb_ref[...]], axis=1)
        hn = _layernorm(h, g_ref[0, 0], b_ref[0, 0])
        hs = hn[:, chalf * (KHALF // 2):(chalf + 1) * (KHALF // 2)]
        basis = _rswaf_gm(hs)
        y = jax.lax.dot_general(basis, w_ref[0], (((1,), (1,)), ((), ())))
        if chalf == 0:
            out_ref[...] = y
        else:
            out_ref[...] = (y + yprev_ref[...]) * ws_ref[...]


def kernel(hidden_states, gate_w, fc1_w, fc1_b, fc2_w, fc2_b,
           ln1_g, ln1_b, sp1_w, ln2_g, ln2_b, sp2_w):
    x = hidden_states
    (tok_map, w_slot, gslot, mlp_eid, mlp_nact,
     kan_eid, kan_nact) = _route_meta(x, gate_w)

    sp1_w = (sp1_w.reshape(HGRP, KHALF, HID, NG)
             .transpose(0, 1, 3, 2).reshape(HGRP, KHALF, HID * NG))
    sp2_w = (sp2_w.reshape(HGRP, HID, 2, KHALF // 2, NG)
             .transpose(0, 1, 2, 4, 3).reshape(HGRP, HID, KHALF * NG))

    fc1_b, fc2_b = fc1_b[:, None, :], fc2_b[:, None, :]
    ln1_g, ln1_b = ln1_g[:, None, :], ln1_b[:, None, :]
    ln2_g, ln2_b = ln2_g[:, None, :], ln2_b[:, None, :]

    xs_all = jnp.take(x, tok_map, axis=0)
    xs_mlp, xs_kan = xs_all[:NSLOT], xs_all[NSLOT:]
    ws_col = w_slot[:, None]
    ws_mlp, ws_kan = ws_col[:NSLOT], ws_col[NSLOT:]

    y_mlp = pl.pallas_call(
        _mlp_body,
        grid_spec=pltpu.PrefetchScalarGridSpec(
            num_scalar_prefetch=2,
            grid=(NBLK,),
            in_specs=[
                pl.BlockSpec((BT, HID), lambda i, e, n: (i, 0)),
                pl.BlockSpec((1, FFN, HID), lambda i, e, n: (e[i], 0, 0)),
                pl.BlockSpec((1, 1, FFN), lambda i, e, n: (e[i], 0, 0)),
                pl.BlockSpec((1, HID, FFN), lambda i, e, n: (e[i], 0, 0)),
                pl.BlockSpec((1, 1, HID), lambda i, e, n: (e[i], 0, 0)),
                pl.BlockSpec((BT, 1), lambda i, e, n: (i, 0)),
            ],
            out_specs=pl.BlockSpec((BT, HID), lambda i, e, n: (i, 0)),
        ),
        out_shape=jax.ShapeDtypeStruct((NSLOT, HID), jnp.float32),
    )(mlp_eid, mlp_nact[None], xs_mlp, fc1_w, fc1_b, fc2_w, fc2_b, ws_mlp)

    def kan1_call(ohalf):
        return pl.pallas_call(
            _kan1_body,
            grid_spec=pltpu.PrefetchScalarGridSpec(
                num_scalar_prefetch=2,
                grid=(NBLK,),
                in_specs=[
                    pl.BlockSpec((BT, HID), lambda i, e, n: (i, 0)),
                    pl.BlockSpec((1, 1, HID), lambda i, e, n: (e[i], 0, 0)),
                    pl.BlockSpec((1, 1, HID), lambda i, e, n: (e[i], 0, 0)),
                    pl.BlockSpec((1, KHALF // 2, HID * NG),
                                 lambda i, e, n: (e[i], ohalf, 0)),
                ],
                out_specs=pl.BlockSpec((BT, KHALF // 2),
                                       lambda i, e, n: (i, 0)),
            ),
            out_shape=jax.ShapeDtypeStruct((NSLOT, KHALF // 2), jnp.float32),
        )(kan_eid, kan_nact[None], xs_kan, ln1_g, ln1_b, sp1_w)

    h_a = kan1_call(0)
    h_b = kan1_call(1)

    def kan2_call(chalf, extra_specs, extra_args):
        return pl.pallas_call(
            functools.partial(_kan2_body, chalf),
            grid_spec=pltpu.PrefetchScalarGridSpec(
                num_scalar_prefetch=2,
                grid=(NBLK,),
                in_specs=[
                    pl.BlockSpec((BT, KHALF // 2), lambda i, e, n: (i, 0)),
                    pl.BlockSpec((BT, KHALF // 2), lambda i, e, n: (i, 0)),
                    pl.BlockSpec((1, 1, KHALF), lambda i, e, n: (e[i], 0, 0)),
                    pl.BlockSpec((1, 1, KHALF), lambda i, e, n: (e[i], 0, 0)),
                    pl.BlockSpec((1, HID, (KHALF // 2) * NG),
                                 lambda i, e, n: (e[i], 0, chalf)),
                ] + extra_specs,
                out_specs=pl.BlockSpec((BT, HID), lambda i, e, n: (i, 0)),
            ),
            out_shape=jax.ShapeDtypeStruct((NSLOT, HID), jnp.float32),
        )(kan_eid, kan_nact[None], h_a, h_b, ln2_g, ln2_b, sp2_w,
          *extra_args)

    y0 = kan2_call(0, [], [])
    y_kan = kan2_call(1, [
        pl.BlockSpec((BT, HID), lambda i, e, n: (i, 0)),
        pl.BlockSpec((BT, 1), lambda i, e, n: (i, 0)),
    ], [y0, ws_kan])

    y_all = jnp.concatenate([y_mlp, y_kan], axis=0)
    out = (jnp.take(y_all, gslot[:, 0], axis=0)
           + jnp.take(y_all, gslot[:, 1], axis=0))
    return out
